# baseline (speedup 1.0000x reference)
import functools

import jax
import jax.numpy as jnp
from jax import lax
from jax.experimental import pallas as pl
from jax.experimental.pallas import tpu as pltpu
from jax.experimental.pallas import tpu_sc as plsc

_NC = 2
_NS = 16
_NW = _NC * _NS
_LANES = 16

_BLK = 128
_LAG = 8

_DT_ROWS = 2000


@functools.lru_cache(maxsize=None)
def _make_tc_detile(n_rows, d):
    assert n_rows % _DT_ROWS == 0 and _DT_ROWS % 2 == 0

    def body(in_ref, out_ref):
        a = in_ref[pl.Slice(0, _DT_ROWS // 2, 2), :]
        b = in_ref[pl.Slice(1, _DT_ROWS // 2, 2), :]
        out_ref[...] = jnp.concatenate([a, b], axis=1)

    return pl.pallas_call(
        body,
        grid=(n_rows // _DT_ROWS,),
        in_specs=[pl.BlockSpec((_DT_ROWS, d), lambda i: (i, 0))],
        out_specs=pl.BlockSpec((_DT_ROWS // 2, 2 * d), lambda i: (i, 0)),
        out_shape=jax.ShapeDtypeStruct((n_rows // 2, 2 * d), jnp.float32),
        compiler_params=pltpu.CompilerParams(
            dimension_semantics=("arbitrary",),
        ),
    )


@functools.lru_cache(maxsize=None)
def _make_sc_gather(n_total, d):
    per_w = n_total // _NW
    n_blocks = per_w // _BLK
    vecs_per_blk = _BLK // _LANES

    mesh = plsc.VectorSubcoreMesh(core_axis_name="c", subcore_axis_name="s")

    @functools.partial(
        pl.kernel,
        mesh=mesh,
        out_type=jax.ShapeDtypeStruct((n_total,), jnp.float32),
        scratch_types=[
            pltpu.VMEM((per_w,), jnp.int32),
            pltpu.VMEM((per_w,), jnp.int32),
            pltpu.VMEM((per_w,), jnp.float32),
            pltpu.SemaphoreType.DMA,
        ],
    )
    def sc_gather(data_hbm, idx_hbm, out_hbm, raw_v, fidx_v, out_v, sem):
        wid = lax.axis_index("s") * _NC + lax.axis_index("c")
        base = wid * per_w

        pltpu.sync_copy(idx_hbm.at[pl.ds(base, per_w)], raw_v)

        def fire(g):
            for v in range(vecs_per_blk):
                off = pl.multiple_of(g * _BLK + v * _LANES, _LANES)
                col = lax.iota(jnp.int32, _LANES) + ((v * _LANES) % d)
                fidx_v[pl.ds(off, _LANES)] = raw_v[pl.ds(off, _LANES)] * d + col
            boff = pl.multiple_of(g * _BLK, _BLK)
            pltpu.async_copy(
                data_hbm.at[fidx_v.at[pl.ds(boff, _BLK)]],
                out_v.at[pl.ds(boff, _BLK)],
                sem,
            )

        def drain(g):
            boff = pl.multiple_of(g * _BLK, _BLK)
            pltpu.make_async_copy(
                data_hbm.at[fidx_v.at[pl.ds(boff, _BLK)]],
                out_v.at[pl.ds(boff, _BLK)],
                sem,
            ).wait()

        def loop_body(g, carry):
            fire(g)

            @pl.when(g >= _LAG)
            def _():
                drain(g - _LAG)

            return carry

        lax.fori_loop(0, n_blocks, loop_body, 0)

        def drain_body(g, carry):
            drain(g)
            return carry

        lax.fori_loop(n_blocks - _LAG, n_blocks, drain_body, 0)

        pltpu.sync_copy(out_v, out_hbm.at[pl.ds(base, per_w)])

    return sc_gather


def kernel(data, indices, axis):
    del axis
    v, d = data.shape
    r, c = indices.shape
    n_total = r * c
    assert c == d
    assert d % _LANES == 0 and _BLK % d == 0
    assert n_total % (_NW * _BLK) == 0

    flat_data = _make_tc_detile(v, d)(data).reshape(v * d)
    flat_idx = indices.reshape(n_total)
    out = _make_sc_gather(n_total, d)(flat_data, flat_idx)
    return out.reshape(r, c)


# speedup vs baseline: 1.0016x; 1.0016x over previous
import functools

import jax
import jax.numpy as jnp
from jax import lax
from jax.experimental import pallas as pl
from jax.experimental.pallas import tpu as pltpu
from jax.experimental.pallas import tpu_sc as plsc

_NC = 2
_NS = 16
_NW = _NC * _NS
_LANES = 16

_BLK = 128
_LAG = 8

_DT_ROWS = 2000


@functools.lru_cache(maxsize=None)
def _make_tc_detile(n_rows, d):
    assert n_rows % _DT_ROWS == 0 and _DT_ROWS % 2 == 0

    def body(in_ref, out_ref):
        a = in_ref[pl.Slice(0, _DT_ROWS // 2, 2), :]
        b = in_ref[pl.Slice(1, _DT_ROWS // 2, 2), :]
        out_ref[...] = jnp.concatenate([a, b], axis=1).reshape(_DT_ROWS * d)

    return pl.pallas_call(
        body,
        grid=(n_rows // _DT_ROWS,),
        in_specs=[pl.BlockSpec((_DT_ROWS, d), lambda i: (i, 0))],
        out_specs=pl.BlockSpec((_DT_ROWS * d,), lambda i: (i,)),
        out_shape=jax.ShapeDtypeStruct((n_rows * d,), jnp.float32),
        compiler_params=pltpu.CompilerParams(
            dimension_semantics=("arbitrary",),
        ),
    )


@functools.lru_cache(maxsize=None)
def _make_sc_gather(n_total, d):
    per_w = n_total // _NW
    n_blocks = per_w // _BLK
    vecs_per_blk = _BLK // _LANES

    mesh = plsc.VectorSubcoreMesh(core_axis_name="c", subcore_axis_name="s")

    @functools.partial(
        pl.kernel,
        mesh=mesh,
        out_type=jax.ShapeDtypeStruct((n_total,), jnp.float32),
        scratch_types=[
            pltpu.VMEM((per_w,), jnp.int32),
            pltpu.VMEM((per_w,), jnp.int32),
            pltpu.VMEM((per_w,), jnp.float32),
            pltpu.SemaphoreType.DMA,
        ],
    )
    def sc_gather(data_hbm, idx_hbm, out_hbm, raw_v, fidx_v, out_v, sem):
        wid = lax.axis_index("s") * _NC + lax.axis_index("c")
        base = wid * per_w

        pltpu.sync_copy(idx_hbm.at[pl.ds(base, per_w)], raw_v)

        def fire(g):
            for v in range(vecs_per_blk):
                off = pl.multiple_of(g * _BLK + v * _LANES, _LANES)
                col = lax.iota(jnp.int32, _LANES) + ((v * _LANES) % d)
                fidx_v[pl.ds(off, _LANES)] = raw_v[pl.ds(off, _LANES)] * d + col
            boff = pl.multiple_of(g * _BLK, _BLK)
            pltpu.async_copy(
                data_hbm.at[fidx_v.at[pl.ds(boff, _BLK)]],
                out_v.at[pl.ds(boff, _BLK)],
                sem,
            )

        def drain(g):
            boff = pl.multiple_of(g * _BLK, _BLK)
            pltpu.make_async_copy(
                data_hbm.at[fidx_v.at[pl.ds(boff, _BLK)]],
                out_v.at[pl.ds(boff, _BLK)],
                sem,
            ).wait()

        def loop_body(g, carry):
            fire(g)

            @pl.when(g >= _LAG)
            def _():
                drain(g - _LAG)

            return carry

        lax.fori_loop(0, n_blocks, loop_body, 0)

        def drain_body(g, carry):
            drain(g)
            return carry

        lax.fori_loop(n_blocks - _LAG, n_blocks, drain_body, 0)

        pltpu.sync_copy(out_v, out_hbm.at[pl.ds(base, per_w)])

    return sc_gather


def kernel(data, indices, axis):
    del axis
    v, d = data.shape
    r, c = indices.shape
    n_total = r * c
    assert c == d
    assert d % _LANES == 0 and _BLK % d == 0
    assert n_total % (_NW * _BLK) == 0

    flat_data = _make_tc_detile(v, d)(data)
    flat_idx = indices.reshape(n_total)
    out = _make_sc_gather(n_total, d)(flat_data, flat_idx)
    return out.reshape(r, c)


# speedup vs baseline: 1.0332x; 1.0316x over previous
import functools

import jax
import jax.numpy as jnp
from jax import lax
from jax.experimental import pallas as pl
from jax.experimental.pallas import tpu as pltpu
from jax.experimental.pallas import tpu_sc as plsc

_NC = 2
_NS = 16
_NW = _NC * _NS
_LANES = 16

_BLK = 128
_LAG = 8

_DT_ROWS = 2000


@functools.lru_cache(maxsize=None)
def _make_tc_detile(n_rows, d):
    assert n_rows % _DT_ROWS == 0 and _DT_ROWS % 2 == 0

    def body(in_ref, out_ref):
        a = in_ref[pl.Slice(0, _DT_ROWS // 2, 2), :]
        b = in_ref[pl.Slice(1, _DT_ROWS // 2, 2), :]
        out_ref[...] = jnp.concatenate([a, b], axis=1).reshape(_DT_ROWS * d)

    return pl.pallas_call(
        body,
        grid=(n_rows // _DT_ROWS,),
        in_specs=[pl.BlockSpec((_DT_ROWS, d), lambda i: (i, 0))],
        out_specs=pl.BlockSpec((_DT_ROWS * d,), lambda i: (i,)),
        out_shape=jax.ShapeDtypeStruct((n_rows * d,), jnp.float32),
        compiler_params=pltpu.CompilerParams(
            dimension_semantics=("arbitrary",),
        ),
    )


@functools.lru_cache(maxsize=None)
def _make_sc_gather(n_total, d):
    per_w = n_total // _NW
    n_blocks = per_w // _BLK
    vecs_per_blk = _BLK // _LANES

    mesh = plsc.VectorSubcoreMesh(core_axis_name="c", subcore_axis_name="s")

    @functools.partial(
        pl.kernel,
        mesh=mesh,
        out_type=jax.ShapeDtypeStruct((n_total,), jnp.float32),
        scratch_types=[
            pltpu.VMEM((per_w,), jnp.int32),
            pltpu.VMEM((per_w,), jnp.int32),
            pltpu.VMEM((per_w,), jnp.float32),
            pltpu.SemaphoreType.DMA,
        ],
    )
    def sc_gather(data_hbm, idx_hbm, out_hbm, raw_v, fidx_v, out_v, sem):
        wid = lax.axis_index("s") * _NC + lax.axis_index("c")
        base = wid * per_w

        pltpu.sync_copy(idx_hbm.at[pl.ds(base, per_w)], raw_v)

        def fire(g):
            for v in range(vecs_per_blk):
                off = pl.multiple_of(g * _BLK + v * _LANES, _LANES)
                col = lax.iota(jnp.int32, _LANES) + ((v * _LANES) % d)
                fidx_v[pl.ds(off, _LANES)] = raw_v[pl.ds(off, _LANES)] * d + col
            boff = pl.multiple_of(g * _BLK, _BLK)
            pltpu.async_copy(
                data_hbm.at[fidx_v.at[pl.ds(boff, _BLK)]],
                out_v.at[pl.ds(boff, _BLK)],
                sem,
            )

        def drain(g):
            boff = pl.multiple_of(g * _BLK, _BLK)
            pltpu.make_async_copy(
                data_hbm.at[fidx_v.at[pl.ds(boff, _BLK)]],
                out_v.at[pl.ds(boff, _BLK)],
                sem,
            ).wait()

        def loop_body(g, carry):
            fire(g)

            @pl.when(g >= _LAG)
            def _():
                drain(g - _LAG)

            return carry

        lax.fori_loop(0, n_blocks, loop_body, 0)

        def drain_body(g, carry):
            drain(g)
            return carry

        lax.fori_loop(n_blocks - _LAG, n_blocks, drain_body, 0)

        pltpu.sync_copy(out_v, out_hbm.at[pl.ds(base, per_w)])

    return sc_gather


def kernel(data, indices, axis):
    del axis
    v, d = data.shape
    r, c = indices.shape
    n_total = r * c
    assert c == d
    assert d % _LANES == 0 and _BLK % d == 0
    assert n_total % (_NW * _BLK) == 0

    pred = indices[0, 0] >= jnp.int32(-1)
    flat_data = jnp.where(pred, data, jnp.float32(0)).reshape(v * d)
    flat_idx = indices.reshape(n_total)
    out = _make_sc_gather(n_total, d)(flat_data, flat_idx)
    return out.reshape(r, c)


# speedup vs baseline: 1.2826x; 1.2413x over previous
import functools

import jax
import jax.numpy as jnp
from jax import lax
from jax.experimental import pallas as pl
from jax.experimental.pallas import tpu as pltpu
from jax.experimental.pallas import tpu_sc as plsc

_NC = 2
_NS = 16
_NW = _NC * _NS
_LANES = 16

_BLK = 128
_LAG = 16


@functools.lru_cache(maxsize=None)
def _make_sc_gather(n_total, d):
    per_w = n_total // _NW
    n_blocks = per_w // _BLK
    vecs_per_blk = _BLK // _LANES

    mesh = plsc.VectorSubcoreMesh(core_axis_name="c", subcore_axis_name="s")

    @functools.partial(
        pl.kernel,
        mesh=mesh,
        out_type=jax.ShapeDtypeStruct((n_total,), jnp.float32),
        scratch_types=[
            pltpu.VMEM((per_w,), jnp.int32),
            pltpu.VMEM((per_w,), jnp.int32),
            pltpu.VMEM((per_w,), jnp.float32),
            pltpu.SemaphoreType.DMA,
        ],
    )
    def sc_gather(data_hbm, idx_hbm, out_hbm, raw_v, fidx_v, out_v, sem):
        wid = lax.axis_index("s") * _NC + lax.axis_index("c")
        base = wid * per_w

        pltpu.sync_copy(idx_hbm.at[pl.ds(base, per_w)], raw_v)

        def fire(g):
            for v in range(vecs_per_blk):
                off = pl.multiple_of(g * _BLK + v * _LANES, _LANES)
                col = lax.iota(jnp.int32, _LANES) + ((v * _LANES) % d)
                fidx_v[pl.ds(off, _LANES)] = raw_v[pl.ds(off, _LANES)] * d + col
            boff = pl.multiple_of(g * _BLK, _BLK)
            pltpu.async_copy(
                data_hbm.at[fidx_v.at[pl.ds(boff, _BLK)]],
                out_v.at[pl.ds(boff, _BLK)],
                sem,
            )

        def drain(g):
            boff = pl.multiple_of(g * _BLK, _BLK)
            pltpu.make_async_copy(
                data_hbm.at[fidx_v.at[pl.ds(boff, _BLK)]],
                out_v.at[pl.ds(boff, _BLK)],
                sem,
            ).wait()

        def loop_body(g, carry):
            fire(g)

            @pl.when(g >= _LAG)
            def _():
                drain(g - _LAG)

            return carry

        lax.fori_loop(0, n_blocks, loop_body, 0)

        def drain_body(g, carry):
            drain(g)
            return carry

        lax.fori_loop(n_blocks - _LAG, n_blocks, drain_body, 0)

        pltpu.sync_copy(out_v, out_hbm.at[pl.ds(base, per_w)])

    return sc_gather


def kernel(data, indices, axis):
    del axis
    v, d = data.shape
    r, c = indices.shape
    n_total = r * c
    assert c == d
    assert d % _LANES == 0 and _BLK % d == 0
    assert n_total % (_NW * _BLK) == 0

    flat_data = data.reshape(v * d)
    flat_idx = indices.reshape(n_total)
    out = _make_sc_gather(n_total, d)(flat_data, flat_idx)
    return out.reshape(r, c)
